# baseline (speedup 1.0000x reference)
import jax
import jax.numpy as jnp
from jax import lax
from jax.experimental import pallas as pl
from jax.experimental.pallas import tpu as pltpu, tpu_sc as plsc

TEMP_INV = 10.0
N = 2048
D = 128
NC = 2
NS = 16
NW = NC * NS
ROWS_PER_TEC = N // NW
NBINS = 13


def _valuation(m):
    v = jnp.zeros((16,), jnp.float32)
    for _ in range(NBINS):
        div = (m > 0) & (lax.rem(m, 3) == 0)
        v = v + jnp.where(div, 1.0, 0.0)
        m = jnp.where(div, lax.div(m, 3), m)
    return v


def _tc_radii_body(z_ref, o_ref):
    x = z_ref[...]
    o_ref[...] = jnp.sqrt(jnp.sum(x * x, axis=-1)) * TEMP_INV


def _sc_body(q_hbm, bidx_hbm, outb_hbm, outc_hbm,
             qfull, idxb, bidx_all, stagev, stagec):
    cid = lax.axis_index("c")
    sid = lax.axis_index("s")
    wid = cid * NS + sid

    pltpu.sync_copy(q_hbm, qfull)

    pltpu.sync_copy(bidx_hbm.at[pl.ds(wid * ROWS_PER_TEC, ROWS_PER_TEC)], idxb)
    vvecs = [_valuation(idxb[pl.ds(k * 16, 16)]) for k in range(4)]
    qvecs = [qfull[pl.ds(wid * ROWS_PER_TEC + k * 16, 16)] for k in range(4)]

    one = jnp.ones((16,), jnp.float32)

    def jstep(t, accs):
        qjv = qfull[pl.ds(t * 16, 16)]
        for dj in range(16):
            qj = jnp.full((16,), qjv[dj])
            accs = tuple(
                acc + one / (one + jnp.exp(qj - qi))
                for acc, qi in zip(accs, qvecs)
            )
        return accs

    zero = jnp.zeros((16,), jnp.float32)
    accs = lax.fori_loop(0, N // 16, jstep, (zero, zero, zero, zero))

    bvec = zero
    for vk, acck in zip(vvecs, accs):
        bvec = bvec + vk * acck
    stagev[...] = bvec
    pltpu.sync_copy(stagev, outb_hbm.at[wid])

    @pl.when(wid == 0)
    def _():
        pltpu.sync_copy(bidx_hbm, bidx_all)

        def hstep(t, cnts):
            v = _valuation(bidx_all[pl.ds(t * 16, 16)])
            return tuple(c + jnp.where(v == float(a), 1.0, 0.0)
                         for a, c in enumerate(cnts))

        cnts = lax.fori_loop(0, N // 16, hstep,
                             tuple(jnp.zeros((16,), jnp.float32)
                                   for _ in range(NBINS)))
        for a in range(NBINS):
            stagec[a, :] = cnts[a]
        for a in range(NBINS, 16):
            stagec[a, :] = jnp.zeros((16,), jnp.float32)
        pltpu.sync_copy(stagec, outc_hbm)


def kernel(z_hyp, batch_indices):
    q = pl.pallas_call(
        _tc_radii_body,
        out_shape=jax.ShapeDtypeStruct((8, N // 8), jnp.float32),
    )(z_hyp.reshape(8, N // 8, D))
    q = q.reshape(N)

    mesh = plsc.VectorSubcoreMesh(core_axis_name="c", subcore_axis_name="s")
    outb, outc = pl.kernel(
        _sc_body,
        mesh=mesh,
        out_type=(
            jax.ShapeDtypeStruct((NW, 16), jnp.float32),
            jax.ShapeDtypeStruct((16, 16), jnp.float32),
        ),
        scratch_types=[
            pltpu.VMEM((N,), jnp.float32),
            pltpu.VMEM((ROWS_PER_TEC,), jnp.int32),
            pltpu.VMEM((N,), jnp.int32),
            pltpu.VMEM((16,), jnp.float32),
            pltpu.VMEM((16, 16), jnp.float32),
        ],
    )(q, batch_indices)

    b_total = jnp.sum(outb)
    c = jnp.sum(outc[:NBINS], axis=1)
    a_idx = jnp.arange(NBINS, dtype=jnp.float32)
    sv = jnp.sum(a_idx * c)
    csq = jnp.sum(c * c)
    a_sum = 0.5 * jnp.sum(c[:, None] * c[None, :]
                          * jnp.abs(a_idx[:, None] - a_idx[None, :]))
    denom = jnp.maximum(float(N * N) - csq, 1.0)
    num = 2.0 * b_total - float(N) * sv + a_sum
    return num / denom


# speedup vs baseline: 2.4953x; 2.4953x over previous
import jax
import jax.numpy as jnp
from jax import lax
from jax.experimental import pallas as pl
from jax.experimental.pallas import tpu as pltpu, tpu_sc as plsc

TEMP_INV = 10.0
N = 2048
D = 128
NC = 2
NS = 16
NW = NC * NS
ROWS_PER_TEC = N // NW
NBINS = 13


INV3 = -1431655765
LIM3 = 0x55555555


def _valuation(m):
    v = jnp.zeros((16,), jnp.float32)
    for _ in range(NBINS):
        t = m * INV3
        div = (m > 0) & (t >= 0) & (t <= LIM3)
        v = v + jnp.where(div, 1.0, 0.0)
        m = jnp.where(div, t, m)
    return v


def _tc_radii_body(z_ref, o_ref):
    x = z_ref[...]
    o_ref[...] = jnp.sqrt(jnp.sum(x * x, axis=-1)) * TEMP_INV


def _sc_body(q_hbm, bidx_hbm, outb_hbm, outc_hbm,
             qfull, idxb, stagev, stagec):
    cid = lax.axis_index("c")
    sid = lax.axis_index("s")
    wid = cid * NS + sid

    pltpu.sync_copy(q_hbm, qfull)

    pltpu.sync_copy(bidx_hbm.at[pl.ds(wid * ROWS_PER_TEC, ROWS_PER_TEC)], idxb)
    vvecs = [_valuation(idxb[pl.ds(k * 16, 16)]) for k in range(4)]
    qvecs = [qfull[pl.ds(wid * ROWS_PER_TEC + k * 16, 16)] for k in range(4)]

    one = jnp.ones((16,), jnp.float32)

    def jstep(t, accs):
        qjv = qfull[pl.ds(t * 16, 16)]
        for dj in range(16):
            qj = jnp.full((16,), qjv[dj])
            accs = tuple(
                acc + one / (one + jnp.exp(qj - qi))
                for acc, qi in zip(accs, qvecs)
            )
        return accs

    zero = jnp.zeros((16,), jnp.float32)
    accs = lax.fori_loop(0, N // 16, jstep, (zero, zero, zero, zero))

    bvec = zero
    for vk, acck in zip(vvecs, accs):
        bvec = bvec + vk * acck
    stagev[...] = bvec
    pltpu.sync_copy(stagev, outb_hbm.at[wid])

    for a in range(NBINS):
        cnt = zero
        for vk in vvecs:
            cnt = cnt + jnp.where(vk == float(a), 1.0, 0.0)
        stagec[a, :] = cnt
    for a in range(NBINS, 16):
        stagec[a, :] = zero
    pltpu.sync_copy(stagec, outc_hbm.at[wid])


def kernel(z_hyp, batch_indices):
    q = pl.pallas_call(
        _tc_radii_body,
        out_shape=jax.ShapeDtypeStruct((8, N // 8), jnp.float32),
    )(z_hyp.reshape(8, N // 8, D))
    q = q.reshape(N)

    mesh = plsc.VectorSubcoreMesh(core_axis_name="c", subcore_axis_name="s")
    outb, outc = pl.kernel(
        _sc_body,
        mesh=mesh,
        out_type=(
            jax.ShapeDtypeStruct((NW, 16), jnp.float32),
            jax.ShapeDtypeStruct((NW, 16, 16), jnp.float32),
        ),
        scratch_types=[
            pltpu.VMEM((N,), jnp.float32),
            pltpu.VMEM((ROWS_PER_TEC,), jnp.int32),
            pltpu.VMEM((16,), jnp.float32),
            pltpu.VMEM((16, 16), jnp.float32),
        ],
    )(q, batch_indices)

    b_total = jnp.sum(outb)
    c = jnp.sum(outc, axis=(0, 2))[:NBINS]
    a_idx = jnp.arange(NBINS, dtype=jnp.float32)
    sv = jnp.sum(a_idx * c)
    csq = jnp.sum(c * c)
    a_sum = 0.5 * jnp.sum(c[:, None] * c[None, :]
                          * jnp.abs(a_idx[:, None] - a_idx[None, :]))
    denom = jnp.maximum(float(N * N) - csq, 1.0)
    num = 2.0 * b_total - float(N) * sv + a_sum
    return num / denom


# speedup vs baseline: 2.8616x; 1.1468x over previous
import jax
import jax.numpy as jnp
from jax import lax
from jax.experimental import pallas as pl
from jax.experimental.pallas import tpu as pltpu, tpu_sc as plsc

TEMP_INV = 10.0
N = 2048
D = 128
NC = 2
NS = 16
NW = NC * NS
ROWS_PER_TEC = N // NW
NBINS = 13


INV3 = -1431655765
LIM3 = 0x55555555


def _valuation(m):
    v = jnp.zeros((16,), jnp.float32)
    for _ in range(NBINS):
        t = m * INV3
        div = (m > 0) & (t >= 0) & (t <= LIM3)
        v = v + jnp.where(div, 1.0, 0.0)
        m = jnp.where(div, t, m)
    return v


def _tc_radii_body(z_ref, o_ref):
    x = z_ref[...]
    q = jnp.sqrt(jnp.sum(x * x, axis=-1)) * TEMP_INV
    o_ref[...] = q - 0.5 * (jnp.max(q) + jnp.min(q))


def _sc_body(q_hbm, bidx_hbm, outb_hbm, outc_hbm,
             qfull, etab, idxb, stagev, stagec):
    cid = lax.axis_index("c")
    sid = lax.axis_index("s")
    wid = cid * NS + sid

    pltpu.sync_copy(q_hbm, qfull)

    pltpu.sync_copy(bidx_hbm.at[pl.ds(wid * ROWS_PER_TEC, ROWS_PER_TEC)], idxb)
    vvecs = [_valuation(idxb[pl.ds(k * 16, 16)]) for k in range(4)]
    qvecs = [qfull[pl.ds(wid * ROWS_PER_TEC + k * 16, 16)] for k in range(4)]

    def estep(t, _):
        qs = qfull[pl.ds(t * 16, 16)]
        etab[pl.ds(t * 16, 16)] = jnp.exp(jnp.clip(qs, -87.0, 87.0))
        return 0

    lax.fori_loop(0, N // 16, estep, 0)
    evecs = [jnp.exp(jnp.clip(-qi, -87.0, 87.0)) for qi in qvecs]

    one = jnp.ones((16,), jnp.float32)

    def jstep(t, accs):
        ejv = etab[pl.ds(t * 16, 16)]
        for dj in range(16):
            ej = jnp.full((16,), ejv[dj])
            accs = tuple(
                acc + one / (one + ej * ei)
                for acc, ei in zip(accs, evecs)
            )
        return accs

    zero = jnp.zeros((16,), jnp.float32)
    accs = lax.fori_loop(0, N // 16, jstep, (zero, zero, zero, zero))

    bvec = zero
    for vk, acck in zip(vvecs, accs):
        bvec = bvec + vk * acck
    stagev[...] = bvec
    pltpu.sync_copy(stagev, outb_hbm.at[wid])

    for a in range(NBINS):
        cnt = zero
        for vk in vvecs:
            cnt = cnt + jnp.where(vk == float(a), 1.0, 0.0)
        stagec[a, :] = cnt
    for a in range(NBINS, 16):
        stagec[a, :] = zero
    pltpu.sync_copy(stagec, outc_hbm.at[wid])


def kernel(z_hyp, batch_indices):
    q = pl.pallas_call(
        _tc_radii_body,
        out_shape=jax.ShapeDtypeStruct((8, N // 8), jnp.float32),
    )(z_hyp.reshape(8, N // 8, D))
    q = q.reshape(N)

    mesh = plsc.VectorSubcoreMesh(core_axis_name="c", subcore_axis_name="s")
    outb, outc = pl.kernel(
        _sc_body,
        mesh=mesh,
        out_type=(
            jax.ShapeDtypeStruct((NW, 16), jnp.float32),
            jax.ShapeDtypeStruct((NW, 16, 16), jnp.float32),
        ),
        scratch_types=[
            pltpu.VMEM((N,), jnp.float32),
            pltpu.VMEM((N,), jnp.float32),
            pltpu.VMEM((ROWS_PER_TEC,), jnp.int32),
            pltpu.VMEM((16,), jnp.float32),
            pltpu.VMEM((16, 16), jnp.float32),
        ],
    )(q, batch_indices)

    b_total = jnp.sum(outb)
    c = jnp.sum(outc, axis=(0, 2))[:NBINS]
    a_idx = jnp.arange(NBINS, dtype=jnp.float32)
    sv = jnp.sum(a_idx * c)
    csq = jnp.sum(c * c)
    a_sum = 0.5 * jnp.sum(c[:, None] * c[None, :]
                          * jnp.abs(a_idx[:, None] - a_idx[None, :]))
    denom = jnp.maximum(float(N * N) - csq, 1.0)
    num = 2.0 * b_total - float(N) * sv + a_sum
    return num / denom
